# baseline (speedup 1.0000x reference)
import math

import jax
import jax.numpy as jnp
import numpy as np
from jax.experimental import pallas as pl
from jax.experimental.pallas import tpu as pltpu

_NUM_TIMESTEPS = 1000


def _schedule_tables():
    steps = _NUM_TIMESTEPS + 1
    x = np.linspace(0, _NUM_TIMESTEPS, steps, dtype=np.float64)
    s = 0.008
    alphas_cumprod = np.cos((x / _NUM_TIMESTEPS + s) / (1 + s) * math.pi * 0.5) ** 2
    alphas_cumprod = alphas_cumprod / alphas_cumprod[0]
    betas = np.clip(1 - alphas_cumprod[1:] / alphas_cumprod[:-1], 0, 0.999)
    ac = np.cumprod(1.0 - betas, axis=0)
    sqrt_ac = np.sqrt(ac).astype(np.float32)
    sqrt_om = np.sqrt(1.0 - ac).astype(np.float32)
    return sqrt_ac, sqrt_om


_SQRT_AC, _SQRT_OM = _schedule_tables()

_NB = 8
_NR = 1


def _body(ts_ref, ta_ref, tb_ref, x_ref, n_ref, o_ref):
    g = pl.program_id(0)
    a = jnp.stack([ta_ref[ts_ref[g * _NB + j]] for j in range(_NB)]).reshape(_NB, 1, 1)
    s = jnp.stack([tb_ref[ts_ref[g * _NB + j]] for j in range(_NB)]).reshape(_NB, 1, 1)
    o_ref[...] = a * x_ref[...] + s * n_ref[...]


def _fma_call(ts, ta, tb, x3, n3, nsteps, in_off, out_blocks):
    B, R, W = x3.shape
    grid_spec = pltpu.PrefetchScalarGridSpec(
        num_scalar_prefetch=3,
        grid=(nsteps,),
        in_specs=[
            pl.BlockSpec((_NB, R, W), lambda b, *_: (b + in_off, 0, 0)),
            pl.BlockSpec((_NB, R, W), lambda b, *_: (b + in_off, 0, 0)),
        ],
        out_specs=pl.BlockSpec((_NB, R, W), lambda b, *_: (b, 0, 0)),
    )
    return pl.pallas_call(
        _body,
        grid_spec=grid_spec,
        out_shape=jax.ShapeDtypeStruct((out_blocks * _NB, R, W), x3.dtype),
    )(ts, ta, tb, x3, n3)


def kernel(x_start, noise, timesteps):
    B, C, H, W = x_start.shape
    R = C * H
    x3 = x_start.reshape(B, R, W)
    n3 = noise.reshape(B, R, W)
    ts = timesteps.astype(jnp.int32)
    ta = jnp.asarray(_SQRT_AC)
    tb = jnp.asarray(_SQRT_OM)

    split = B - _NB
    nmain = split // _NB
    out_main = _fma_call(ts, ta, tb, x3, n3, nmain, 0, B // _NB)
    out_tail = _fma_call(ts[split:], ta, tb, x3, n3, 1, nmain, 1)
    out = jax.lax.dynamic_update_slice(out_main, out_tail, (split, 0, 0))
    return out.reshape(B, C, H, W)


# speedup vs baseline: 1.2303x; 1.2303x over previous
import math

import jax
import jax.numpy as jnp
import numpy as np
from jax.experimental import pallas as pl
from jax.experimental.pallas import tpu as pltpu

_NUM_TIMESTEPS = 1000


def _schedule_tables():
    steps = _NUM_TIMESTEPS + 1
    x = np.linspace(0, _NUM_TIMESTEPS, steps, dtype=np.float64)
    s = 0.008
    alphas_cumprod = np.cos((x / _NUM_TIMESTEPS + s) / (1 + s) * math.pi * 0.5) ** 2
    alphas_cumprod = alphas_cumprod / alphas_cumprod[0]
    betas = np.clip(1 - alphas_cumprod[1:] / alphas_cumprod[:-1], 0, 0.999)
    ac = np.cumprod(1.0 - betas, axis=0)
    sqrt_ac = np.sqrt(ac).astype(np.float32)
    sqrt_om = np.sqrt(1.0 - ac).astype(np.float32)
    return sqrt_ac, sqrt_om


_SQRT_AC, _SQRT_OM = _schedule_tables()

_NB = 8
_NR = 1


def _body(ts_ref, ta_ref, tb_ref, x_ref, n_ref, o_ref):
    g = pl.program_id(0)
    a = jnp.stack([ta_ref[ts_ref[g * _NB + j]] for j in range(_NB)]).reshape(_NB, 1, 1)
    s = jnp.stack([tb_ref[ts_ref[g * _NB + j]] for j in range(_NB)]).reshape(_NB, 1, 1)
    o_ref[...] = a * x_ref[...] + s * n_ref[...]


def kernel(x_start, noise, timesteps):
    B, C, H, W = x_start.shape
    R = C * H
    x3 = x_start.reshape(B, R, W)
    n3 = noise.reshape(B, R, W)
    ts = timesteps.astype(jnp.int32)
    ta = jnp.asarray(_SQRT_AC)
    tb = jnp.asarray(_SQRT_OM)
    rblk = R // _NR

    grid_spec = pltpu.PrefetchScalarGridSpec(
        num_scalar_prefetch=3,
        grid=(B // _NB, _NR),
        in_specs=[
            pl.BlockSpec((_NB, rblk, W), lambda b, r, *_: (b, r, 0)),
            pl.BlockSpec((_NB, rblk, W), lambda b, r, *_: (b, r, 0)),
        ],
        out_specs=pl.BlockSpec((_NB, rblk, W), lambda b, r, *_: (b, r, 0)),
    )
    out = pl.pallas_call(
        _body,
        grid_spec=grid_spec,
        out_shape=jax.ShapeDtypeStruct((B, R, W), x_start.dtype),
    )(ts, ta, tb, x3, n3)
    return out.reshape(B, C, H, W)
